# baseline (speedup 1.0000x reference)
import functools

import jax
import jax.numpy as jnp
from jax import lax
from jax.experimental import pallas as pl
from jax.experimental.pallas import tpu as pltpu
from jax.experimental.pallas import tpu_sc as plsc

NC, NS, L = 2, 16, 16
NW = NC * NS
GRID = 128
GRID2 = GRID * GRID
GRID3 = GRID * GRID * GRID
CHUNK = 128
GROUPS = CHUNK // L


@functools.lru_cache(maxsize=None)
def _sc_feat_kernel(M):
    PTSW = M // NW
    NCHUNK = PTSW // CHUNK
    mesh = plsc.VectorSubcoreMesh(core_axis_name="c", subcore_axis_name="s")

    @functools.partial(
        pl.kernel,
        out_type=jax.ShapeDtypeStruct((M * 3,), jnp.float32),
        mesh=mesh,
        scratch_types=[
            pltpu.VMEM((PTSW,), jnp.float32),
            pltpu.VMEM((PTSW,), jnp.float32),
            pltpu.VMEM((PTSW,), jnp.float32),
            pltpu.VMEM((2, 24, CHUNK), jnp.int32),
            pltpu.VMEM((2, 24, CHUNK), jnp.float32),
            pltpu.VMEM((2, 4, CHUNK), jnp.float32),
            pltpu.VMEM((2, 3, CHUNK), jnp.float32),
            pltpu.SemaphoreType.DMA,
            pltpu.SemaphoreType.DMA,
            pltpu.SemaphoreType.DMA,
            pltpu.SemaphoreType.DMA,
        ],
    )
    def k(px_h, py_h, pz_h, tab_h, out_h, px_v, py_v, pz_v, idx_v, gat_v,
          w_v, feat_v, semA, semB, semF0, semF1):
        wid = lax.axis_index("s") * NC + lax.axis_index("c")
        base = wid * PTSW
        pltpu.sync_copy(px_h.at[pl.ds(base, PTSW)], px_v)
        pltpu.sync_copy(py_h.at[pl.ds(base, PTSW)], py_v)
        pltpu.sync_copy(pz_h.at[pl.ds(base, PTSW)], pz_v)

        def compute_idx(off, sl):
            for j in range(GROUPS):
                s = off + j * L
                px = px_v[pl.ds(s, L)]
                py = py_v[pl.ds(s, L)]
                pz = pz_v[pl.ds(s, L)]
                in_f = ((px > -1.0) & (px < 1.0) & (py > -1.0) & (py < 1.0)
                        & (pz > -1.0) & (pz < 1.0))
                in_b = ((px > -4.0) & (px < 4.0) & (py > -4.0) & (py < 4.0)
                        & (pz > -4.0) & (pz < 4.0))
                sc = jnp.where(in_f, 1.0, 0.25)
                half = 0.5 * (GRID - 1)
                fx = jnp.clip((pz * sc + 1.0) * half, 0.0, GRID - 1.0)
                fy = jnp.clip((py * sc + 1.0) * half, 0.0, GRID - 1.0)
                fz = jnp.clip((px * sc + 1.0) * half, 0.0, GRID - 1.0)
                x0 = fx.astype(jnp.int32)
                y0 = fy.astype(jnp.int32)
                z0 = fz.astype(jnp.int32)
                wx = fx - x0.astype(jnp.float32)
                wy = fy - y0.astype(jnp.float32)
                wz = fz - z0.astype(jnp.float32)
                x1 = jnp.minimum(x0 + 1, GRID - 1)
                y1 = jnp.minimum(y0 + 1, GRID - 1)
                z1 = jnp.minimum(z0 + 1, GRID - 1)
                basei = jnp.where(in_f, 0, 3 * GRID3)
                za = z0 * GRID2 + basei
                zb = z1 * GRID2 + basei
                ya = y0 * GRID
                yb = y1 * GRID
                for t in range(8):
                    vox = ((zb if (t & 4) else za) + (yb if (t & 2) else ya)
                           + (x1 if (t & 1) else x0))
                    for c3 in range(3):
                        idx_v[sl, c3 * 8 + t, pl.ds(j * L, L)] = vox + (c3 * GRID3)
                w_v[sl, 0, pl.ds(j * L, L)] = wx
                w_v[sl, 1, pl.ds(j * L, L)] = wy
                w_v[sl, 2, pl.ds(j * L, L)] = wz
                w_v[sl, 3, pl.ds(j * L, L)] = jnp.where(in_f | in_b, 1.0, 0.0)

        def fire(sl, sem):
            for r in range(24):
                pltpu.async_copy(tab_h.at[idx_v.at[sl, r]], gat_v.at[sl, r], sem)

        def drain(sl, sem):
            for r in range(24):
                pltpu.make_async_copy(tab_h.at[idx_v.at[sl, r]],
                                      gat_v.at[sl, r], sem).wait()

        def combine(off, sl, semF, first=False):
            if not first:
                for c3 in range(3):
                    pltpu.make_async_copy(
                        feat_v.at[sl, c3],
                        out_h.at[pl.ds(c3 * M + base, CHUNK)], semF).wait()
            for j in range(GROUPS):
                wx = w_v[sl, 0, pl.ds(j * L, L)]
                wy = w_v[sl, 1, pl.ds(j * L, L)]
                wz = w_v[sl, 2, pl.ds(j * L, L)]
                inb = w_v[sl, 3, pl.ds(j * L, L)]
                w = []
                for t in range(8):
                    wt = ((wz if (t & 4) else (1.0 - wz))
                          * (wy if (t & 2) else (1.0 - wy))
                          * (wx if (t & 1) else (1.0 - wx)))
                    w.append(wt)
                for c3 in range(3):
                    acc = gat_v[sl, c3 * 8 + 0, pl.ds(j * L, L)] * w[0]
                    for t in range(1, 8):
                        acc = acc + gat_v[sl, c3 * 8 + t, pl.ds(j * L, L)] * w[t]
                    acc = jnp.where(inb > 0.5, acc, 0.5)
                    feat_v[sl, c3, pl.ds(j * L, L)] = acc
            for c3 in range(3):
                pltpu.async_copy(feat_v.at[sl, c3],
                                 out_h.at[pl.ds(c3 * M + base + off, CHUNK)],
                                 semF)

        compute_idx(0, 0)
        fire(0, semA)
        compute_idx(CHUNK, 1)
        fire(1, semB)
        drain(0, semA)
        combine(0, 0, semF0, first=True)
        compute_idx(2 * CHUNK, 0)
        fire(0, semA)
        drain(1, semB)
        combine(CHUNK, 1, semF1, first=True)

        def pair_body(i, carry):
            off0 = i * (2 * CHUNK)
            compute_idx(off0 + CHUNK, 1)
            fire(1, semB)
            drain(0, semA)
            combine(off0, 0, semF0)
            compute_idx(off0 + 2 * CHUNK, 0)
            fire(0, semA)
            drain(1, semB)
            combine(off0 + CHUNK, 1, semF1)
            return carry

        lax.fori_loop(1, NCHUNK // 2 - 1, pair_body, 0)

        off0 = (NCHUNK - 2) * CHUNK
        compute_idx(off0 + CHUNK, 1)
        fire(1, semB)
        drain(0, semA)
        combine(off0, 0, semF0)
        drain(1, semB)
        combine(off0 + CHUNK, 1, semF1)
        for sl, semF in ((0, semF0), (1, semF1)):
            for c3 in range(3):
                pltpu.make_async_copy(
                    feat_v.at[sl, c3],
                    out_h.at[pl.ds(c3 * M + base, CHUNK)], semF).wait()

    return k


def _mlp_body(feat_ref, v_ref, w1a_ref, w2t_ref, b2_ref, out_ref):
    P = feat_ref.shape[1]
    one = 1.0 + 0.0 * pl.program_id(0).astype(jnp.float32)
    emb = jnp.concatenate([feat_ref[...], v_ref[0],
                           jnp.broadcast_to(one, (1, P))], axis=0)
    h = jnp.dot(w1a_ref[...], emb, preferred_element_type=jnp.float32)
    h = jnp.maximum(h, 0.0).astype(jnp.bfloat16)
    out_ref[...] = (jnp.dot(w2t_ref[...], h, preferred_element_type=jnp.float32)
                    + b2_ref[...])


@functools.lru_cache(maxsize=None)
def _mlp_kernel(M, N, H):
    P = 2048
    nb = N // P
    return pl.pallas_call(
        _mlp_body,
        grid=(M // P,),
        in_specs=[
            pl.BlockSpec((3, P), lambda i: (0, i)),
            pl.BlockSpec((1, 2, P), lambda i: (i // nb, 0, i % nb)),
            pl.BlockSpec((H, 6), lambda i: (0, 0)),
            pl.BlockSpec((3, H), lambda i: (0, 0)),
            pl.BlockSpec((3, 1), lambda i: (0, 0)),
        ],
        out_specs=pl.BlockSpec((3, P), lambda i: (0, i)),
        out_shape=jax.ShapeDtypeStruct((3, M), jnp.float32),
    )


def kernel(x_i, v, fg_feat, bg_feat, W_in, b_in, W_out, b_out):
    B, N = x_i.shape[0], x_i.shape[1]
    M = B * N
    H = W_in.shape[1]
    p = x_i.reshape(M, 3)
    table = jnp.concatenate([fg_feat.reshape(-1), bg_feat.reshape(-1)])
    v_fixed = v.at[0, :, 0].set(-v[0, :, 0]).at[0, :, 1].set(jnp.pi - v[0, :, 1])
    S = 4
    SM = M // S
    SB = B // S
    px, py, pz = p[:, 0], p[:, 1], p[:, 2]
    w1a = jnp.concatenate([W_in.T, b_in.reshape(H, 1)], axis=1)
    w2t = W_out.T.astype(jnp.bfloat16)
    b2c = b_out.reshape(3, 1)
    feats = [_sc_feat_kernel(SM)(px[i * SM:(i + 1) * SM],
                                 py[i * SM:(i + 1) * SM],
                                 pz[i * SM:(i + 1) * SM], table)
             for i in range(S)]
    outs = [_mlp_kernel(SM, N, H)(feats[i].reshape(3, SM),
                                  v_fixed[i * SB:(i + 1) * SB], w1a, w2t, b2c)
            for i in range(S)]
    out_cm = jnp.concatenate(outs, axis=1)
    return out_cm.T.reshape(B, N, 3)


# speedup vs baseline: 1.0464x; 1.0464x over previous
import functools

import jax
import jax.numpy as jnp
from jax import lax
from jax.experimental import pallas as pl
from jax.experimental.pallas import tpu as pltpu
from jax.experimental.pallas import tpu_sc as plsc

NC, NS, L = 2, 16, 16
NW = NC * NS
GRID = 128
GRID2 = GRID * GRID
GRID3 = GRID * GRID * GRID
CHUNK = 128
GROUPS = CHUNK // L


@functools.lru_cache(maxsize=None)
def _sc_feat_kernel(M):
    PTSW = M // NW
    NCHUNK = PTSW // CHUNK
    mesh = plsc.VectorSubcoreMesh(core_axis_name="c", subcore_axis_name="s")

    @functools.partial(
        pl.kernel,
        out_type=jax.ShapeDtypeStruct((M * 3,), jnp.float32),
        mesh=mesh,
        scratch_types=[
            pltpu.VMEM((PTSW,), jnp.float32),
            pltpu.VMEM((PTSW,), jnp.float32),
            pltpu.VMEM((PTSW,), jnp.float32),
            pltpu.VMEM((2, 24, CHUNK), jnp.int32),
            pltpu.VMEM((2, 24, CHUNK), jnp.float32),
            pltpu.VMEM((2, 4, CHUNK), jnp.float32),
            pltpu.VMEM((2, 3, CHUNK), jnp.float32),
            pltpu.SemaphoreType.DMA,
            pltpu.SemaphoreType.DMA,
            pltpu.SemaphoreType.DMA,
            pltpu.SemaphoreType.DMA,
        ],
    )
    def k(px_h, py_h, pz_h, tab_h, out_h, px_v, py_v, pz_v, idx_v, gat_v,
          w_v, feat_v, semA, semB, semF0, semF1):
        wid = lax.axis_index("s") * NC + lax.axis_index("c")
        base = wid * PTSW
        pltpu.sync_copy(px_h.at[pl.ds(base, PTSW)], px_v)
        pltpu.sync_copy(py_h.at[pl.ds(base, PTSW)], py_v)
        pltpu.sync_copy(pz_h.at[pl.ds(base, PTSW)], pz_v)

        def compute_idx(off, sl):
            for j in range(GROUPS):
                s = off + j * L
                px = px_v[pl.ds(s, L)]
                py = py_v[pl.ds(s, L)]
                pz = pz_v[pl.ds(s, L)]
                in_f = ((px > -1.0) & (px < 1.0) & (py > -1.0) & (py < 1.0)
                        & (pz > -1.0) & (pz < 1.0))
                in_b = ((px > -4.0) & (px < 4.0) & (py > -4.0) & (py < 4.0)
                        & (pz > -4.0) & (pz < 4.0))
                sc = jnp.where(in_f, 1.0, 0.25)
                half = 0.5 * (GRID - 1)
                fx = jnp.clip((pz * sc + 1.0) * half, 0.0, GRID - 1.0)
                fy = jnp.clip((py * sc + 1.0) * half, 0.0, GRID - 1.0)
                fz = jnp.clip((px * sc + 1.0) * half, 0.0, GRID - 1.0)
                x0 = fx.astype(jnp.int32)
                y0 = fy.astype(jnp.int32)
                z0 = fz.astype(jnp.int32)
                wx = fx - x0.astype(jnp.float32)
                wy = fy - y0.astype(jnp.float32)
                wz = fz - z0.astype(jnp.float32)
                x1 = jnp.minimum(x0 + 1, GRID - 1)
                y1 = jnp.minimum(y0 + 1, GRID - 1)
                z1 = jnp.minimum(z0 + 1, GRID - 1)
                basei = jnp.where(in_f, 0, 3 * GRID3)
                za = z0 * GRID2 + basei
                zb = z1 * GRID2 + basei
                ya = y0 * GRID
                yb = y1 * GRID
                for t in range(8):
                    vox = ((zb if (t & 4) else za) + (yb if (t & 2) else ya)
                           + (x1 if (t & 1) else x0))
                    for c3 in range(3):
                        idx_v[sl, c3 * 8 + t, pl.ds(j * L, L)] = vox + (c3 * GRID3)
                w_v[sl, 0, pl.ds(j * L, L)] = wx
                w_v[sl, 1, pl.ds(j * L, L)] = wy
                w_v[sl, 2, pl.ds(j * L, L)] = wz
                w_v[sl, 3, pl.ds(j * L, L)] = jnp.where(in_f | in_b, 1.0, 0.0)

        def fire(sl, sem):
            for r in range(24):
                pltpu.async_copy(tab_h.at[idx_v.at[sl, r]], gat_v.at[sl, r], sem)

        def drain(sl, sem):
            for r in range(24):
                pltpu.make_async_copy(tab_h.at[idx_v.at[sl, r]],
                                      gat_v.at[sl, r], sem).wait()

        def combine(off, sl, semF, first=False):
            if not first:
                for c3 in range(3):
                    pltpu.make_async_copy(
                        feat_v.at[sl, c3],
                        out_h.at[pl.ds(c3 * M + base, CHUNK)], semF).wait()
            for j in range(GROUPS):
                wx = w_v[sl, 0, pl.ds(j * L, L)]
                wy = w_v[sl, 1, pl.ds(j * L, L)]
                wz = w_v[sl, 2, pl.ds(j * L, L)]
                inb = w_v[sl, 3, pl.ds(j * L, L)]
                w = []
                for t in range(8):
                    wt = ((wz if (t & 4) else (1.0 - wz))
                          * (wy if (t & 2) else (1.0 - wy))
                          * (wx if (t & 1) else (1.0 - wx)))
                    w.append(wt)
                for c3 in range(3):
                    acc = gat_v[sl, c3 * 8 + 0, pl.ds(j * L, L)] * w[0]
                    for t in range(1, 8):
                        acc = acc + gat_v[sl, c3 * 8 + t, pl.ds(j * L, L)] * w[t]
                    acc = jnp.where(inb > 0.5, acc, 0.5)
                    feat_v[sl, c3, pl.ds(j * L, L)] = acc
            for c3 in range(3):
                pltpu.async_copy(feat_v.at[sl, c3],
                                 out_h.at[pl.ds(c3 * M + base + off, CHUNK)],
                                 semF)

        compute_idx(0, 0)
        fire(0, semA)
        compute_idx(CHUNK, 1)
        fire(1, semB)
        drain(0, semA)
        combine(0, 0, semF0, first=True)
        compute_idx(2 * CHUNK, 0)
        fire(0, semA)
        drain(1, semB)
        combine(CHUNK, 1, semF1, first=True)

        def pair_body(i, carry):
            off0 = i * (2 * CHUNK)
            compute_idx(off0 + CHUNK, 1)
            fire(1, semB)
            drain(0, semA)
            combine(off0, 0, semF0)
            compute_idx(off0 + 2 * CHUNK, 0)
            fire(0, semA)
            drain(1, semB)
            combine(off0 + CHUNK, 1, semF1)
            return carry

        lax.fori_loop(1, NCHUNK // 2 - 1, pair_body, 0)

        off0 = (NCHUNK - 2) * CHUNK
        compute_idx(off0 + CHUNK, 1)
        fire(1, semB)
        drain(0, semA)
        combine(off0, 0, semF0)
        drain(1, semB)
        combine(off0 + CHUNK, 1, semF1)
        for sl, semF in ((0, semF0), (1, semF1)):
            for c3 in range(3):
                pltpu.make_async_copy(
                    feat_v.at[sl, c3],
                    out_h.at[pl.ds(c3 * M + base, CHUNK)], semF).wait()

    return k


def _mlp_body(feat_ref, v_ref, w1a_ref, w2t_ref, b2_ref, out_ref):
    P = feat_ref.shape[1]
    one = 1.0 + 0.0 * pl.program_id(0).astype(jnp.float32)
    emb = jnp.concatenate([feat_ref[...], v_ref[0],
                           jnp.broadcast_to(one, (1, P))], axis=0)
    h = jnp.dot(w1a_ref[...], emb, preferred_element_type=jnp.float32)
    h = jnp.maximum(h, 0.0).astype(jnp.bfloat16)
    out_ref[...] = (jnp.dot(w2t_ref[...], h, preferred_element_type=jnp.float32)
                    + b2_ref[...])


@functools.lru_cache(maxsize=None)
def _mlp_kernel(M, N, H):
    P = 2048
    nb = N // P
    return pl.pallas_call(
        _mlp_body,
        grid=(M // P,),
        in_specs=[
            pl.BlockSpec((3, P), lambda i: (0, i)),
            pl.BlockSpec((1, 2, P), lambda i: (i // nb, 0, i % nb)),
            pl.BlockSpec((H, 6), lambda i: (0, 0)),
            pl.BlockSpec((3, H), lambda i: (0, 0)),
            pl.BlockSpec((3, 1), lambda i: (0, 0)),
        ],
        out_specs=pl.BlockSpec((3, P), lambda i: (0, i)),
        out_shape=jax.ShapeDtypeStruct((3, M), jnp.float32),
    )


def kernel(x_i, v, fg_feat, bg_feat, W_in, b_in, W_out, b_out):
    B, N = x_i.shape[0], x_i.shape[1]
    M = B * N
    H = W_in.shape[1]
    p = x_i.reshape(M, 3)
    table = jnp.concatenate([fg_feat.reshape(-1), bg_feat.reshape(-1)])
    v_fixed = v.at[0, :, 0].set(-v[0, :, 0]).at[0, :, 1].set(jnp.pi - v[0, :, 1])
    HM = M // 2
    HB = B // 2
    px, py, pz = p[:, 0], p[:, 1], p[:, 2]
    w1a = jnp.concatenate([W_in.T, b_in.reshape(H, 1)], axis=1)
    w2t = W_out.T.astype(jnp.bfloat16)
    b2c = b_out.reshape(3, 1)
    feat_a = _sc_feat_kernel(HM)(px[:HM], py[:HM], pz[:HM], table)
    feat_b = _sc_feat_kernel(HM)(px[HM:], py[HM:], pz[HM:], table)
    out_a = _mlp_kernel(HM, N, H)(feat_a.reshape(3, HM), v_fixed[:HB], w1a,
                                  w2t, b2c)
    out_b = _mlp_kernel(HM, N, H)(feat_b.reshape(3, HM), v_fixed[HB:], w1a,
                                  w2t, b2c)
    out_cm = jnp.concatenate([out_a, out_b], axis=1)
    return out_cm.T.reshape(B, N, 3)


# speedup vs baseline: 1.0551x; 1.0083x over previous
import functools

import jax
import jax.numpy as jnp
from jax import lax
from jax.experimental import pallas as pl
from jax.experimental.pallas import tpu as pltpu
from jax.experimental.pallas import tpu_sc as plsc

NC, NS, L = 2, 16, 16
NW = NC * NS
GRID = 128
GRID2 = GRID * GRID
GRID3 = GRID * GRID * GRID
CHUNK = 128
GROUPS = CHUNK // L


@functools.lru_cache(maxsize=None)
def _sc_feat_kernel(M):
    PTSW = M // NW
    NCHUNK = PTSW // CHUNK
    mesh = plsc.VectorSubcoreMesh(core_axis_name="c", subcore_axis_name="s")

    @functools.partial(
        pl.kernel,
        out_type=jax.ShapeDtypeStruct((M * 3,), jnp.float32),
        mesh=mesh,
        scratch_types=[
            pltpu.VMEM((PTSW,), jnp.float32),
            pltpu.VMEM((PTSW,), jnp.float32),
            pltpu.VMEM((PTSW,), jnp.float32),
            pltpu.VMEM((2, 24, CHUNK), jnp.int32),
            pltpu.VMEM((2, 24, CHUNK), jnp.float32),
            pltpu.VMEM((2, 4, CHUNK), jnp.float32),
            pltpu.VMEM((2, 3, CHUNK), jnp.float32),
            pltpu.SemaphoreType.DMA,
            pltpu.SemaphoreType.DMA,
            pltpu.SemaphoreType.DMA,
            pltpu.SemaphoreType.DMA,
        ],
    )
    def k(px_h, py_h, pz_h, tab_h, out_h, px_v, py_v, pz_v, idx_v, gat_v,
          w_v, feat_v, semA, semB, semF0, semF1):
        wid = lax.axis_index("s") * NC + lax.axis_index("c")
        base = wid * PTSW
        pltpu.sync_copy(px_h.at[pl.ds(base, PTSW)], px_v)
        pltpu.sync_copy(py_h.at[pl.ds(base, PTSW)], py_v)
        pltpu.sync_copy(pz_h.at[pl.ds(base, PTSW)], pz_v)

        def compute_idx(off, sl):
            for j in range(GROUPS):
                s = off + j * L
                px = px_v[pl.ds(s, L)]
                py = py_v[pl.ds(s, L)]
                pz = pz_v[pl.ds(s, L)]
                in_f = ((px > -1.0) & (px < 1.0) & (py > -1.0) & (py < 1.0)
                        & (pz > -1.0) & (pz < 1.0))
                in_b = ((px > -4.0) & (px < 4.0) & (py > -4.0) & (py < 4.0)
                        & (pz > -4.0) & (pz < 4.0))
                sc = jnp.where(in_f, 1.0, 0.25)
                half = 0.5 * (GRID - 1)
                fx = jnp.clip((pz * sc + 1.0) * half, 0.0, GRID - 1.0)
                fy = jnp.clip((py * sc + 1.0) * half, 0.0, GRID - 1.0)
                fz = jnp.clip((px * sc + 1.0) * half, 0.0, GRID - 1.0)
                x0 = fx.astype(jnp.int32)
                y0 = fy.astype(jnp.int32)
                z0 = fz.astype(jnp.int32)
                wx = fx - x0.astype(jnp.float32)
                wy = fy - y0.astype(jnp.float32)
                wz = fz - z0.astype(jnp.float32)
                x1 = jnp.minimum(x0 + 1, GRID - 1)
                y1 = jnp.minimum(y0 + 1, GRID - 1)
                z1 = jnp.minimum(z0 + 1, GRID - 1)
                basei = jnp.where(in_f, 0, 3 * GRID3)
                za = z0 * GRID2 + basei
                zb = z1 * GRID2 + basei
                ya = y0 * GRID
                yb = y1 * GRID
                for t in range(8):
                    vox = ((zb if (t & 4) else za) + (yb if (t & 2) else ya)
                           + (x1 if (t & 1) else x0))
                    for c3 in range(3):
                        idx_v[sl, c3 * 8 + t, pl.ds(j * L, L)] = vox + (c3 * GRID3)
                w_v[sl, 0, pl.ds(j * L, L)] = wx
                w_v[sl, 1, pl.ds(j * L, L)] = wy
                w_v[sl, 2, pl.ds(j * L, L)] = wz
                w_v[sl, 3, pl.ds(j * L, L)] = jnp.where(in_f | in_b, 1.0, 0.0)

        def fire(sl, sem):
            for r in range(24):
                pltpu.async_copy(tab_h.at[idx_v.at[sl, r]], gat_v.at[sl, r], sem)

        def drain(sl, sem):
            for r in range(24):
                pltpu.make_async_copy(tab_h.at[idx_v.at[sl, r]],
                                      gat_v.at[sl, r], sem).wait()

        def combine(off, sl, semF, first=False):
            if not first:
                for c3 in range(3):
                    pltpu.make_async_copy(
                        feat_v.at[sl, c3],
                        out_h.at[pl.ds(c3 * M + base, CHUNK)], semF).wait()
            for j in range(GROUPS):
                wx = w_v[sl, 0, pl.ds(j * L, L)]
                wy = w_v[sl, 1, pl.ds(j * L, L)]
                wz = w_v[sl, 2, pl.ds(j * L, L)]
                inb = w_v[sl, 3, pl.ds(j * L, L)]
                w = []
                for t in range(8):
                    wt = ((wz if (t & 4) else (1.0 - wz))
                          * (wy if (t & 2) else (1.0 - wy))
                          * (wx if (t & 1) else (1.0 - wx)))
                    w.append(wt)
                for c3 in range(3):
                    acc = gat_v[sl, c3 * 8 + 0, pl.ds(j * L, L)] * w[0]
                    for t in range(1, 8):
                        acc = acc + gat_v[sl, c3 * 8 + t, pl.ds(j * L, L)] * w[t]
                    acc = jnp.where(inb > 0.5, acc, 0.5)
                    feat_v[sl, c3, pl.ds(j * L, L)] = acc
            for c3 in range(3):
                pltpu.async_copy(feat_v.at[sl, c3],
                                 out_h.at[pl.ds(c3 * M + base + off, CHUNK)],
                                 semF)

        compute_idx(0, 0)
        fire(0, semA)
        compute_idx(CHUNK, 1)
        fire(1, semB)
        drain(0, semA)
        combine(0, 0, semF0, first=True)
        compute_idx(2 * CHUNK, 0)
        fire(0, semA)
        drain(1, semB)
        combine(CHUNK, 1, semF1, first=True)

        def pair_body(i, carry):
            off0 = i * (2 * CHUNK)
            compute_idx(off0 + CHUNK, 1)
            fire(1, semB)
            drain(0, semA)
            combine(off0, 0, semF0)
            compute_idx(off0 + 2 * CHUNK, 0)
            fire(0, semA)
            drain(1, semB)
            combine(off0 + CHUNK, 1, semF1)
            return carry

        lax.fori_loop(1, NCHUNK // 2 - 1, pair_body, 0)

        off0 = (NCHUNK - 2) * CHUNK
        compute_idx(off0 + CHUNK, 1)
        fire(1, semB)
        drain(0, semA)
        combine(off0, 0, semF0)
        drain(1, semB)
        combine(off0 + CHUNK, 1, semF1)
        for sl, semF in ((0, semF0), (1, semF1)):
            for c3 in range(3):
                pltpu.make_async_copy(
                    feat_v.at[sl, c3],
                    out_h.at[pl.ds(c3 * M + base, CHUNK)], semF).wait()

    return k


def _mlp_body(feat_ref, v_ref, w1a_ref, w2t_ref, b2_ref, out_ref):
    P = feat_ref.shape[1]
    one = 1.0 + 0.0 * pl.program_id(0).astype(jnp.float32)
    emb = jnp.concatenate([feat_ref[...], v_ref[0],
                           jnp.broadcast_to(one, (1, P))], axis=0)
    h = jnp.dot(w1a_ref[...], emb, preferred_element_type=jnp.float32)
    h = jnp.maximum(h, 0.0).astype(jnp.bfloat16)
    out_ref[...] = (jnp.dot(w2t_ref[...], h, preferred_element_type=jnp.float32)
                    + b2_ref[...])


@functools.lru_cache(maxsize=None)
def _mlp_kernel(M, N, H):
    P = 4096
    nb = N // P
    return pl.pallas_call(
        _mlp_body,
        grid=(M // P,),
        in_specs=[
            pl.BlockSpec((3, P), lambda i: (0, i)),
            pl.BlockSpec((1, 2, P), lambda i: (i // nb, 0, i % nb)),
            pl.BlockSpec((H, 6), lambda i: (0, 0)),
            pl.BlockSpec((3, H), lambda i: (0, 0)),
            pl.BlockSpec((3, 1), lambda i: (0, 0)),
        ],
        out_specs=pl.BlockSpec((3, P), lambda i: (0, i)),
        out_shape=jax.ShapeDtypeStruct((3, M), jnp.float32),
    )


def kernel(x_i, v, fg_feat, bg_feat, W_in, b_in, W_out, b_out):
    B, N = x_i.shape[0], x_i.shape[1]
    M = B * N
    H = W_in.shape[1]
    p = x_i.reshape(M, 3)
    table = jnp.concatenate([fg_feat.reshape(-1), bg_feat.reshape(-1)])
    v_fixed = v.at[0, :, 0].set(-v[0, :, 0]).at[0, :, 1].set(jnp.pi - v[0, :, 1])
    HM = M // 2
    HB = B // 2
    px, py, pz = p[:, 0], p[:, 1], p[:, 2]
    w1a = jnp.concatenate([W_in.T, b_in.reshape(H, 1)], axis=1)
    w2t = W_out.T.astype(jnp.bfloat16)
    b2c = b_out.reshape(3, 1)
    feat_a = _sc_feat_kernel(HM)(px[:HM], py[:HM], pz[:HM], table)
    feat_b = _sc_feat_kernel(HM)(px[HM:], py[HM:], pz[HM:], table)
    out_a = _mlp_kernel(HM, N, H)(feat_a.reshape(3, HM), v_fixed[:HB], w1a,
                                  w2t, b2c)
    out_b = _mlp_kernel(HM, N, H)(feat_b.reshape(3, HM), v_fixed[HB:], w1a,
                                  w2t, b2c)
    out_cm = jnp.concatenate([out_a, out_b], axis=1)
    return out_cm.T.reshape(B, N, 3)


# speedup vs baseline: 1.0597x; 1.0043x over previous
import functools

import jax
import jax.numpy as jnp
from jax import lax
from jax.experimental import pallas as pl
from jax.experimental.pallas import tpu as pltpu
from jax.experimental.pallas import tpu_sc as plsc

NC, NS, L = 2, 16, 16
NW = NC * NS
GRID = 128
GRID2 = GRID * GRID
GRID3 = GRID * GRID * GRID
CHUNK = 128
GROUPS = CHUNK // L


@functools.lru_cache(maxsize=None)
def _sc_feat_kernel(M):
    PTSW = M // NW
    NCHUNK = PTSW // CHUNK
    mesh = plsc.VectorSubcoreMesh(core_axis_name="c", subcore_axis_name="s")

    @functools.partial(
        pl.kernel,
        out_type=jax.ShapeDtypeStruct((M * 3,), jnp.float32),
        mesh=mesh,
        scratch_types=[
            pltpu.VMEM((PTSW,), jnp.float32),
            pltpu.VMEM((PTSW,), jnp.float32),
            pltpu.VMEM((PTSW,), jnp.float32),
            pltpu.VMEM((2, 24, CHUNK), jnp.int32),
            pltpu.VMEM((2, 24, CHUNK), jnp.float32),
            pltpu.VMEM((2, 4, CHUNK), jnp.float32),
            pltpu.VMEM((2, 3, CHUNK), jnp.float32),
            pltpu.SemaphoreType.DMA,
            pltpu.SemaphoreType.DMA,
            pltpu.SemaphoreType.DMA,
            pltpu.SemaphoreType.DMA,
        ],
    )
    def k(px_h, py_h, pz_h, tab_h, out_h, px_v, py_v, pz_v, idx_v, gat_v,
          w_v, feat_v, semA, semB, semF0, semF1):
        wid = lax.axis_index("s") * NC + lax.axis_index("c")
        base = wid * PTSW
        pltpu.sync_copy(px_h.at[pl.ds(base, PTSW)], px_v)
        pltpu.sync_copy(py_h.at[pl.ds(base, PTSW)], py_v)
        pltpu.sync_copy(pz_h.at[pl.ds(base, PTSW)], pz_v)

        def compute_idx(off, sl):
            for j in range(GROUPS):
                s = off + j * L
                px = px_v[pl.ds(s, L)]
                py = py_v[pl.ds(s, L)]
                pz = pz_v[pl.ds(s, L)]
                in_f = ((px > -1.0) & (px < 1.0) & (py > -1.0) & (py < 1.0)
                        & (pz > -1.0) & (pz < 1.0))
                in_b = ((px > -4.0) & (px < 4.0) & (py > -4.0) & (py < 4.0)
                        & (pz > -4.0) & (pz < 4.0))
                sc = jnp.where(in_f, 1.0, 0.25)
                half = 0.5 * (GRID - 1)
                fx = jnp.clip((pz * sc + 1.0) * half, 0.0, GRID - 1.0)
                fy = jnp.clip((py * sc + 1.0) * half, 0.0, GRID - 1.0)
                fz = jnp.clip((px * sc + 1.0) * half, 0.0, GRID - 1.0)
                x0 = fx.astype(jnp.int32)
                y0 = fy.astype(jnp.int32)
                z0 = fz.astype(jnp.int32)
                wx = fx - x0.astype(jnp.float32)
                wy = fy - y0.astype(jnp.float32)
                wz = fz - z0.astype(jnp.float32)
                x1 = jnp.minimum(x0 + 1, GRID - 1)
                y1 = jnp.minimum(y0 + 1, GRID - 1)
                z1 = jnp.minimum(z0 + 1, GRID - 1)
                basei = jnp.where(in_f, 0, 3 * GRID3)
                za = z0 * GRID2 + basei
                zb = z1 * GRID2 + basei
                ya = y0 * GRID
                yb = y1 * GRID
                for t in range(8):
                    vox = ((zb if (t & 4) else za) + (yb if (t & 2) else ya)
                           + (x1 if (t & 1) else x0))
                    for c3 in range(3):
                        idx_v[sl, c3 * 8 + t, pl.ds(j * L, L)] = vox + (c3 * GRID3)
                w_v[sl, 0, pl.ds(j * L, L)] = wx
                w_v[sl, 1, pl.ds(j * L, L)] = wy
                w_v[sl, 2, pl.ds(j * L, L)] = wz
                w_v[sl, 3, pl.ds(j * L, L)] = jnp.where(in_f | in_b, 1.0, 0.0)

        def fire(sl, sem):
            for r in range(24):
                pltpu.async_copy(tab_h.at[idx_v.at[sl, r]], gat_v.at[sl, r], sem)

        def drain(sl, sem):
            for r in range(24):
                pltpu.make_async_copy(tab_h.at[idx_v.at[sl, r]],
                                      gat_v.at[sl, r], sem).wait()

        def combine(off, sl, semF, first=False):
            if not first:
                for c3 in range(3):
                    pltpu.make_async_copy(
                        feat_v.at[sl, c3],
                        out_h.at[pl.ds(c3 * M + base, CHUNK)], semF).wait()
            for j in range(GROUPS):
                wx = w_v[sl, 0, pl.ds(j * L, L)]
                wy = w_v[sl, 1, pl.ds(j * L, L)]
                wz = w_v[sl, 2, pl.ds(j * L, L)]
                inb = w_v[sl, 3, pl.ds(j * L, L)]
                w = []
                for t in range(8):
                    wt = ((wz if (t & 4) else (1.0 - wz))
                          * (wy if (t & 2) else (1.0 - wy))
                          * (wx if (t & 1) else (1.0 - wx)))
                    w.append(wt)
                for c3 in range(3):
                    acc = gat_v[sl, c3 * 8 + 0, pl.ds(j * L, L)] * w[0]
                    for t in range(1, 8):
                        acc = acc + gat_v[sl, c3 * 8 + t, pl.ds(j * L, L)] * w[t]
                    acc = jnp.where(inb > 0.5, acc, 0.5)
                    feat_v[sl, c3, pl.ds(j * L, L)] = acc
            for c3 in range(3):
                pltpu.async_copy(feat_v.at[sl, c3],
                                 out_h.at[pl.ds(c3 * M + base + off, CHUNK)],
                                 semF)

        compute_idx(0, 0)
        fire(0, semA)
        compute_idx(CHUNK, 1)
        fire(1, semB)
        drain(0, semA)
        combine(0, 0, semF0, first=True)
        compute_idx(2 * CHUNK, 0)
        fire(0, semA)
        drain(1, semB)
        combine(CHUNK, 1, semF1, first=True)

        def pair_body(i, carry):
            off0 = i * (2 * CHUNK)
            compute_idx(off0 + CHUNK, 1)
            fire(1, semB)
            drain(0, semA)
            combine(off0, 0, semF0)
            compute_idx(off0 + 2 * CHUNK, 0)
            fire(0, semA)
            drain(1, semB)
            combine(off0 + CHUNK, 1, semF1)
            return carry

        lax.fori_loop(1, NCHUNK // 2 - 1, pair_body, 0)

        off0 = (NCHUNK - 2) * CHUNK
        compute_idx(off0 + CHUNK, 1)
        fire(1, semB)
        drain(0, semA)
        combine(off0, 0, semF0)
        drain(1, semB)
        combine(off0 + CHUNK, 1, semF1)
        for sl, semF in ((0, semF0), (1, semF1)):
            for c3 in range(3):
                pltpu.make_async_copy(
                    feat_v.at[sl, c3],
                    out_h.at[pl.ds(c3 * M + base, CHUNK)], semF).wait()

    return k


def _mlp_body(feat_ref, v_ref, w1a_ref, w2t_ref, b2_ref, out_ref):
    P = feat_ref.shape[1]
    one = 1.0 + 0.0 * pl.program_id(0).astype(jnp.float32)
    emb = jnp.concatenate([feat_ref[...], v_ref[0],
                           jnp.broadcast_to(one, (1, P))], axis=0)
    h = jnp.dot(w1a_ref[...], emb, preferred_element_type=jnp.float32)
    h = jnp.maximum(h, 0.0).astype(jnp.bfloat16)
    out_ref[...] = (jnp.dot(w2t_ref[...], h, preferred_element_type=jnp.float32)
                    + b2_ref[...])


@functools.lru_cache(maxsize=None)
def _mlp_kernel(M, N, H):
    P = 8192
    nb = N // P
    return pl.pallas_call(
        _mlp_body,
        grid=(M // P,),
        in_specs=[
            pl.BlockSpec((3, P), lambda i: (0, i)),
            pl.BlockSpec((1, 2, P), lambda i: (i // nb, 0, i % nb)),
            pl.BlockSpec((H, 6), lambda i: (0, 0)),
            pl.BlockSpec((3, H), lambda i: (0, 0)),
            pl.BlockSpec((3, 1), lambda i: (0, 0)),
        ],
        out_specs=pl.BlockSpec((3, P), lambda i: (0, i)),
        out_shape=jax.ShapeDtypeStruct((3, M), jnp.float32),
    )


def kernel(x_i, v, fg_feat, bg_feat, W_in, b_in, W_out, b_out):
    B, N = x_i.shape[0], x_i.shape[1]
    M = B * N
    H = W_in.shape[1]
    p = x_i.reshape(M, 3)
    table = jnp.concatenate([fg_feat.reshape(-1), bg_feat.reshape(-1)])
    v_fixed = v.at[0, :, 0].set(-v[0, :, 0]).at[0, :, 1].set(jnp.pi - v[0, :, 1])
    HM = M // 2
    HB = B // 2
    px, py, pz = p[:, 0], p[:, 1], p[:, 2]
    w1a = jnp.concatenate([W_in.T, b_in.reshape(H, 1)], axis=1)
    w2t = W_out.T.astype(jnp.bfloat16)
    b2c = b_out.reshape(3, 1)
    feat_a = _sc_feat_kernel(HM)(px[:HM], py[:HM], pz[:HM], table)
    feat_b = _sc_feat_kernel(HM)(px[HM:], py[HM:], pz[HM:], table)
    out_a = _mlp_kernel(HM, N, H)(feat_a.reshape(3, HM), v_fixed[:HB], w1a,
                                  w2t, b2c)
    out_b = _mlp_kernel(HM, N, H)(feat_b.reshape(3, HM), v_fixed[HB:], w1a,
                                  w2t, b2c)
    out_cm = jnp.concatenate([out_a, out_b], axis=1)
    return out_cm.T.reshape(B, N, 3)
